# baseline (speedup 1.0000x reference)
import jax
import jax.numpy as jnp
from jax import lax
from jax.experimental import pallas as pl
from jax.experimental.pallas import tpu as pltpu

_B = 8
_N = 1024
_K = 33
_H = (_K - 1) // 2
_RBLK = 512
_CBLK = 128
_PADC = 1e30


def _twt_body(v_ref, d_ref, wt_ref, wa_ref, ts_ref, tip_ref, coef_ref):
    v = v_ref[...]
    d = d_ref[...]
    dz = d[:, 1:] - d[:, :-1]
    inv_v = 1.0 / v
    inv_mid = 0.5 * (inv_v[:, :-1] + inv_v[:, 1:])
    dtwt = (2.0 * dz) * inv_mid
    k_iota = lax.broadcasted_iota(jnp.int32, (_N - 1, _N), 0)
    j_iota = lax.broadcasted_iota(jnp.int32, (_N - 1, _N), 1)
    tri = (k_iota < j_iota).astype(jnp.float32)
    ts = lax.dot_general(dtwt, tri, (((1,), (0,)), ((), ())),
                         precision=lax.Precision.HIGHEST,
                         preferred_element_type=jnp.float32)
    ts_ref[...] = ts
    ti = 0.5 * (ts[:, :-1] + ts[:, 1:])
    tip_ref[...] = jnp.concatenate(
        [ti, jnp.full((_B, 1), _PADC, jnp.float32)], axis=1)
    wt = wt_ref[...]
    wa = wa_ref[...]
    eps = jnp.finfo(jnp.float32).eps
    u = wt[:, _H:]
    a = wa[:, _H:]
    s = (a[:, 1:] - a[:, :-1]) / jnp.maximum(u[:, 1:] - u[:, :-1], eps)
    g = s[:, 1:] - s[:, :-1]
    b0 = a[:, 0:1] - s[:, 0:1] * u[:, 0:1]
    coef_ref[...] = jnp.zeros((16, 128), jnp.float32)
    coef_ref[0:1, 0:_H + 1] = u
    coef_ref[1:2, 0:1] = s[:, 0:1]
    coef_ref[1:2, 1:_H] = g
    coef_ref[2:3, 0:1] = b0
    tip_full = tip_ref[...]
    for i in range(_N // _RBLK):
        blk = ts[:, i * _RBLK:(i + 1) * _RBLK]
        coef_ref[8:16, i:i + 1] = jnp.min(blk, axis=1, keepdims=True)
        coef_ref[8:16, 8 + i:9 + i] = jnp.max(blk, axis=1, keepdims=True)
    for t in range(_N // _CBLK):
        blk = tip_full[:, t * _CBLK:(t + 1) * _CBLK]
        coef_ref[8:16, 16 + t:17 + t] = jnp.min(blk, axis=1, keepdims=True)
        coef_ref[8:16, 24 + t:25 + t] = jnp.max(blk, axis=1, keepdims=True)


def _band_body(coef_ref, tsc_ref, tip_ref, out_ref):
    b = pl.program_id(0)
    i = pl.program_id(1)
    u_knots = [coef_ref[0, k] for k in range(_H + 1)]
    g_coefs = [coef_ref[1, k] for k in range(_H)]
    b0 = coef_ref[2, 0]
    u_last = u_knots[_H]
    rmin = coef_ref[8 + b, i]
    rmax = coef_ref[8 + b, 8 + i]
    tsb = jnp.broadcast_to(tsc_ref[0], (_RBLK, _CBLK))
    nt = _N // _CBLK

    def _interp(ti):
        atau = jnp.abs(tsb - ti)
        accs = [b0 + g_coefs[0] * atau,
                g_coefs[1] * jnp.maximum(atau - u_knots[1], 0.0),
                g_coefs[2] * jnp.maximum(atau - u_knots[2], 0.0)]
        for k in range(3, _H):
            accs[k % 3] = accs[k % 3] + g_coefs[k] * jnp.maximum(
                atau - u_knots[k], 0.0)
        acc = (accs[0] + accs[1]) + accs[2]
        return jnp.where(atau > u_last, 0.0, acc)

    out_ref[0] = jnp.zeros((_RBLK, _N - 1), jnp.float32)

    lives = []
    for t in range(nt):
        cmin = coef_ref[8 + b, 16 + t]
        cmax = coef_ref[8 + b, 24 + t]
        lives.append(jnp.logical_and(rmax - cmin >= -u_last,
                                     rmin - cmax <= u_last))
    t_lo = jnp.int32(nt)
    t_hi = jnp.int32(0)
    for t in range(nt):
        t_lo = jnp.minimum(t_lo, jnp.where(lives[t], t, nt))
        t_hi = jnp.maximum(t_hi, jnp.where(lives[t], t + 1, 0))

    def _tile(t, _):
        lo = t * _CBLK
        ti = tip_ref[0, :, pl.ds(lo, _CBLK)]
        out_ref[0, :, pl.ds(lo, _CBLK)] = _interp(ti)
        return _

    lax.fori_loop(t_lo, jnp.minimum(t_hi, nt - 1), _tile, None)

    @pl.when(lives[nt - 1])
    def _tail():
        lo = (nt - 1) * _CBLK
        val = _interp(tip_ref[0, :, lo:lo + _CBLK])
        out_ref[0, :, lo:_N - 1] = val[:, :_CBLK - 1]


def kernel(velocity_mps, depth_axis_m, wavelet_time_s, wavelet_amp):
    v = velocity_mps.astype(jnp.float32)
    d = depth_axis_m.astype(jnp.float32).reshape(1, _N)
    wt = wavelet_time_s.astype(jnp.float32).reshape(1, _K)
    wa = wavelet_amp.astype(jnp.float32).reshape(1, _K)

    ts, tip, coef = pl.pallas_call(
        _twt_body,
        out_shape=(
            jax.ShapeDtypeStruct((_B, _N), jnp.float32),
            jax.ShapeDtypeStruct((_B, _N), jnp.float32),
            jax.ShapeDtypeStruct((16, 128), jnp.float32),
        ),
    )(v, d, wt, wa)

    tsc = ts.reshape(_B, _N, 1)
    tip3 = tip.reshape(_B, 1, _N)
    nr = _N // _RBLK

    return pl.pallas_call(
        _band_body,
        grid=(_B, nr),
        in_specs=[
            pl.BlockSpec(memory_space=pltpu.SMEM),
            pl.BlockSpec((1, _RBLK, 1), lambda b, i: (b, i, 0)),
            pl.BlockSpec((1, 1, _N), lambda b, i: (b, 0, 0)),
        ],
        out_specs=pl.BlockSpec((1, _RBLK, _N - 1), lambda b, i: (b, i, 0)),
        out_shape=jax.ShapeDtypeStruct((_B, _N, _N - 1), jnp.float32),
        compiler_params=pltpu.CompilerParams(
            dimension_semantics=("parallel", "parallel")),
    )(coef, tsc, tip3)


# speedup vs baseline: 1.0215x; 1.0215x over previous
import jax
import jax.numpy as jnp
from jax import lax
from jax.experimental import pallas as pl
from jax.experimental.pallas import tpu as pltpu

_B = 8
_N = 1024
_K = 33
_H = (_K - 1) // 2
_RBLK = 128
_CBLK = 128
_PADC = 1e30


def _twt_body(v_ref, d_ref, wt_ref, wa_ref, ts_ref, tip_ref, coef_ref):
    v = v_ref[...]
    d = d_ref[...]
    dz = d[:, 1:] - d[:, :-1]
    inv_v = 1.0 / v
    inv_mid = 0.5 * (inv_v[:, :-1] + inv_v[:, 1:])
    dtwt = (2.0 * dz) * inv_mid
    k_iota = lax.broadcasted_iota(jnp.int32, (_N - 1, _N), 0)
    j_iota = lax.broadcasted_iota(jnp.int32, (_N - 1, _N), 1)
    tri = (k_iota < j_iota).astype(jnp.float32)
    ts = lax.dot_general(dtwt, tri, (((1,), (0,)), ((), ())),
                         precision=lax.Precision.HIGHEST,
                         preferred_element_type=jnp.float32)
    ts_ref[...] = ts
    ti = 0.5 * (ts[:, :-1] + ts[:, 1:])
    tip_ref[...] = jnp.concatenate(
        [ti, jnp.full((_B, 1), _PADC, jnp.float32)], axis=1)
    wt = wt_ref[...]
    wa = wa_ref[...]
    eps = jnp.finfo(jnp.float32).eps
    u = wt[:, _H:]
    a = wa[:, _H:]
    s = (a[:, 1:] - a[:, :-1]) / jnp.maximum(u[:, 1:] - u[:, :-1], eps)
    g = s[:, 1:] - s[:, :-1]
    b0 = a[:, 0:1] - s[:, 0:1] * u[:, 0:1]
    coef_ref[...] = jnp.zeros((16, 128), jnp.float32)
    coef_ref[0:1, 0:_H + 1] = u
    coef_ref[1:2, 0:1] = s[:, 0:1]
    coef_ref[1:2, 1:_H] = g
    coef_ref[2:3, 0:1] = b0
    tip_full = tip_ref[...]
    for i in range(_N // _RBLK):
        blk = ts[:, i * _RBLK:(i + 1) * _RBLK]
        coef_ref[8:16, i:i + 1] = jnp.min(blk, axis=1, keepdims=True)
        coef_ref[8:16, 8 + i:9 + i] = jnp.max(blk, axis=1, keepdims=True)
    for t in range(_N // _CBLK):
        blk = tip_full[:, t * _CBLK:(t + 1) * _CBLK]
        coef_ref[8:16, 16 + t:17 + t] = jnp.min(blk, axis=1, keepdims=True)
        coef_ref[8:16, 24 + t:25 + t] = jnp.max(blk, axis=1, keepdims=True)


def _band_body(coef_ref, tsc_ref, tip_ref, out_ref):
    b = pl.program_id(0)
    i = pl.program_id(1)
    u_knots = [coef_ref[0, k] for k in range(_H + 1)]
    g_coefs = [coef_ref[1, k] for k in range(_H)]
    b0 = coef_ref[2, 0]
    u_last = u_knots[_H]
    rmin = coef_ref[8 + b, i]
    rmax = coef_ref[8 + b, 8 + i]
    tsb = jnp.broadcast_to(tsc_ref[0], (_RBLK, _CBLK))
    nt = _N // _CBLK

    def _interp(ti):
        atau = jnp.abs(tsb - ti)
        accs = [b0 + g_coefs[0] * atau,
                g_coefs[1] * jnp.maximum(atau - u_knots[1], 0.0),
                g_coefs[2] * jnp.maximum(atau - u_knots[2], 0.0)]
        for k in range(3, _H):
            accs[k % 3] = accs[k % 3] + g_coefs[k] * jnp.maximum(
                atau - u_knots[k], 0.0)
        acc = (accs[0] + accs[1]) + accs[2]
        return jnp.where(atau > u_last, 0.0, acc)

    out_ref[0] = jnp.zeros((_RBLK, _N - 1), jnp.float32)

    lives = []
    for t in range(nt):
        cmin = coef_ref[8 + b, 16 + t]
        cmax = coef_ref[8 + b, 24 + t]
        lives.append(jnp.logical_and(rmax - cmin >= -u_last,
                                     rmin - cmax <= u_last))
    t_lo = jnp.int32(nt)
    t_hi = jnp.int32(0)
    for t in range(nt):
        t_lo = jnp.minimum(t_lo, jnp.where(lives[t], t, nt))
        t_hi = jnp.maximum(t_hi, jnp.where(lives[t], t + 1, 0))

    def _tile(t, _):
        lo = t * _CBLK
        ti = tip_ref[0, :, pl.ds(lo, _CBLK)]
        out_ref[0, :, pl.ds(lo, _CBLK)] = _interp(ti)
        return _

    lax.fori_loop(t_lo, jnp.minimum(t_hi, nt - 1), _tile, None)

    @pl.when(lives[nt - 1])
    def _tail():
        lo = (nt - 1) * _CBLK
        val = _interp(tip_ref[0, :, lo:lo + _CBLK])
        out_ref[0, :, lo:_N - 1] = val[:, :_CBLK - 1]


def kernel(velocity_mps, depth_axis_m, wavelet_time_s, wavelet_amp):
    v = velocity_mps.astype(jnp.float32)
    d = depth_axis_m.astype(jnp.float32).reshape(1, _N)
    wt = wavelet_time_s.astype(jnp.float32).reshape(1, _K)
    wa = wavelet_amp.astype(jnp.float32).reshape(1, _K)

    ts, tip, coef = pl.pallas_call(
        _twt_body,
        out_shape=(
            jax.ShapeDtypeStruct((_B, _N), jnp.float32),
            jax.ShapeDtypeStruct((_B, _N), jnp.float32),
            jax.ShapeDtypeStruct((16, 128), jnp.float32),
        ),
    )(v, d, wt, wa)

    tsc = ts.reshape(_B, _N, 1)
    tip3 = tip.reshape(_B, 1, _N)
    nr = _N // _RBLK

    return pl.pallas_call(
        _band_body,
        grid=(_B, nr),
        in_specs=[
            pl.BlockSpec(memory_space=pltpu.SMEM),
            pl.BlockSpec((1, _RBLK, 1), lambda b, i: (b, i, 0)),
            pl.BlockSpec((1, 1, _N), lambda b, i: (b, 0, 0)),
        ],
        out_specs=pl.BlockSpec((1, _RBLK, _N - 1), lambda b, i: (b, i, 0)),
        out_shape=jax.ShapeDtypeStruct((_B, _N, _N - 1), jnp.float32),
        compiler_params=pltpu.CompilerParams(
            dimension_semantics=("parallel", "parallel")),
    )(coef, tsc, tip3)


# speedup vs baseline: 1.1168x; 1.0933x over previous
import jax
import jax.numpy as jnp
from jax import lax
from jax.experimental import pallas as pl
from jax.experimental.pallas import tpu as pltpu

_B = 8
_N = 1024
_K = 33
_H = (_K - 1) // 2
_RBLK = 256
_CBLK = 128
_PADC = 1e30


def _twt_body(v_ref, d_ref, wt_ref, wa_ref, ts_ref, tip_ref, coef_ref):
    v = v_ref[...]
    d = d_ref[...]
    dz = d[:, 1:] - d[:, :-1]
    inv_v = 1.0 / v
    inv_mid = 0.5 * (inv_v[:, :-1] + inv_v[:, 1:])
    dtwt = (2.0 * dz) * inv_mid
    k_iota = lax.broadcasted_iota(jnp.int32, (_N - 1, _N), 0)
    j_iota = lax.broadcasted_iota(jnp.int32, (_N - 1, _N), 1)
    tri = (k_iota < j_iota).astype(jnp.float32)
    ts = lax.dot_general(dtwt, tri, (((1,), (0,)), ((), ())),
                         precision=lax.Precision.HIGHEST,
                         preferred_element_type=jnp.float32)
    ts_ref[...] = ts
    ti = 0.5 * (ts[:, :-1] + ts[:, 1:])
    tip_ref[...] = jnp.concatenate(
        [ti, jnp.full((_B, 1), _PADC, jnp.float32)], axis=1)
    wt = wt_ref[...]
    wa = wa_ref[...]
    eps = jnp.finfo(jnp.float32).eps
    u = wt[:, _H:]
    a = wa[:, _H:]
    s = (a[:, 1:] - a[:, :-1]) / jnp.maximum(u[:, 1:] - u[:, :-1], eps)
    g = s[:, 1:] - s[:, :-1]
    b0 = a[:, 0:1] - s[:, 0:1] * u[:, 0:1]
    coef_ref[...] = jnp.zeros((16, 128), jnp.float32)
    coef_ref[0:1, 0:_H + 1] = u
    coef_ref[1:2, 0:1] = s[:, 0:1]
    coef_ref[1:2, 1:_H] = g
    coef_ref[2:3, 0:1] = b0
    tip_full = tip_ref[...]
    for i in range(_N // _RBLK):
        blk = ts[:, i * _RBLK:(i + 1) * _RBLK]
        coef_ref[8:16, i:i + 1] = jnp.min(blk, axis=1, keepdims=True)
        coef_ref[8:16, 8 + i:9 + i] = jnp.max(blk, axis=1, keepdims=True)
    for t in range(_N // _CBLK):
        blk = tip_full[:, t * _CBLK:(t + 1) * _CBLK]
        coef_ref[8:16, 16 + t:17 + t] = jnp.min(blk, axis=1, keepdims=True)
        coef_ref[8:16, 24 + t:25 + t] = jnp.max(blk, axis=1, keepdims=True)


def _band_body(coef_ref, tsc_ref, tip_ref, out_ref):
    b = pl.program_id(0)
    i = pl.program_id(1)
    u_knots = [coef_ref[0, k] for k in range(_H + 1)]
    g_coefs = [coef_ref[1, k] for k in range(_H)]
    b0 = coef_ref[2, 0]
    u_last = u_knots[_H]
    rmin = coef_ref[8 + b, i]
    rmax = coef_ref[8 + b, 8 + i]
    tsc = tsc_ref[0, pl.ds(i * _RBLK, _RBLK), :]
    tsb = jnp.broadcast_to(tsc, (_RBLK, _CBLK))
    nt = _N // _CBLK

    def _interp(ti):
        atau = jnp.abs(tsb - ti)
        accs = [b0 + g_coefs[0] * atau,
                g_coefs[1] * jnp.maximum(atau - u_knots[1], 0.0),
                g_coefs[2] * jnp.maximum(atau - u_knots[2], 0.0)]
        for k in range(3, _H):
            accs[k % 3] = accs[k % 3] + g_coefs[k] * jnp.maximum(
                atau - u_knots[k], 0.0)
        acc = (accs[0] + accs[1]) + accs[2]
        return jnp.where(atau > u_last, 0.0, acc)

    out_ref[0] = jnp.zeros((_RBLK, _N - 1), jnp.float32)

    lives = []
    for t in range(nt):
        cmin = coef_ref[8 + b, 16 + t]
        cmax = coef_ref[8 + b, 24 + t]
        lives.append(jnp.logical_and(rmax - cmin >= -u_last,
                                     rmin - cmax <= u_last))
    t_lo = jnp.int32(nt)
    t_hi = jnp.int32(0)
    for t in range(nt):
        t_lo = jnp.minimum(t_lo, jnp.where(lives[t], t, nt))
        t_hi = jnp.maximum(t_hi, jnp.where(lives[t], t + 1, 0))

    def _tile(t, _):
        lo = t * _CBLK
        ti = tip_ref[0, :, pl.ds(lo, _CBLK)]
        out_ref[0, :, pl.ds(lo, _CBLK)] = _interp(ti)
        return _

    lax.fori_loop(t_lo, jnp.minimum(t_hi, nt - 1), _tile, None)

    @pl.when(lives[nt - 1])
    def _tail():
        lo = (nt - 1) * _CBLK
        val = _interp(tip_ref[0, :, lo:lo + _CBLK])
        out_ref[0, :, lo:_N - 1] = val[:, :_CBLK - 1]


def kernel(velocity_mps, depth_axis_m, wavelet_time_s, wavelet_amp):
    v = velocity_mps.astype(jnp.float32)
    d = depth_axis_m.astype(jnp.float32).reshape(1, _N)
    wt = wavelet_time_s.astype(jnp.float32).reshape(1, _K)
    wa = wavelet_amp.astype(jnp.float32).reshape(1, _K)

    ts, tip, coef = pl.pallas_call(
        _twt_body,
        out_shape=(
            jax.ShapeDtypeStruct((_B, _N), jnp.float32),
            jax.ShapeDtypeStruct((_B, _N), jnp.float32),
            jax.ShapeDtypeStruct((16, 128), jnp.float32),
        ),
    )(v, d, wt, wa)

    tsc = ts.reshape(_B, _N, 1)
    tip3 = tip.reshape(_B, 1, _N)
    nr = _N // _RBLK

    return pl.pallas_call(
        _band_body,
        grid=(_B, nr),
        in_specs=[
            pl.BlockSpec(memory_space=pltpu.SMEM),
            pl.BlockSpec((1, _N, 1), lambda b, i: (b, 0, 0)),
            pl.BlockSpec((1, 1, _N), lambda b, i: (b, 0, 0)),
        ],
        out_specs=pl.BlockSpec((1, _RBLK, _N - 1), lambda b, i: (b, i, 0)),
        out_shape=jax.ShapeDtypeStruct((_B, _N, _N - 1), jnp.float32),
        compiler_params=pltpu.CompilerParams(
            dimension_semantics=("parallel", "parallel")),
    )(coef, tsc, tip3)


# speedup vs baseline: 1.1455x; 1.0257x over previous
import jax
import jax.numpy as jnp
from jax import lax
from jax.experimental import pallas as pl
from jax.experimental.pallas import tpu as pltpu

_B = 8
_N = 1024
_K = 33
_H = (_K - 1) // 2
_RBLK = 256
_CBLK = 128
_PADC = 1e30


def _twt_body(v_ref, d_ref, wt_ref, wa_ref, ts_ref, tip_ref, coef_ref):
    v = v_ref[...]
    d = d_ref[...]
    dz = d[:, 1:] - d[:, :-1]
    inv_v = 1.0 / v
    inv_mid = 0.5 * (inv_v[:, :-1] + inv_v[:, 1:])
    dtwt = (2.0 * dz) * inv_mid
    k_iota = lax.broadcasted_iota(jnp.int32, (_N - 1, _N), 0)
    j_iota = lax.broadcasted_iota(jnp.int32, (_N - 1, _N), 1)
    tri = (k_iota < j_iota).astype(jnp.float32)
    ts = lax.dot_general(dtwt, tri, (((1,), (0,)), ((), ())),
                         precision=lax.Precision.HIGHEST,
                         preferred_element_type=jnp.float32)
    ts_ref[...] = ts
    ti = 0.5 * (ts[:, :-1] + ts[:, 1:])
    tip_ref[...] = jnp.concatenate(
        [ti, jnp.full((_B, 1), _PADC, jnp.float32)], axis=1)
    wt = wt_ref[...]
    wa = wa_ref[...]
    eps = jnp.finfo(jnp.float32).eps
    u = wt[:, _H:]
    a = wa[:, _H:]
    s = (a[:, 1:] - a[:, :-1]) / jnp.maximum(u[:, 1:] - u[:, :-1], eps)
    g = s[:, 1:] - s[:, :-1]
    b0 = a[:, 0:1] - s[:, 0:1] * u[:, 0:1]
    coef_ref[...] = jnp.zeros((16, 128), jnp.float32)
    coef_ref[0:1, 0:_H + 1] = u
    coef_ref[1:2, 0:1] = s[:, 0:1]
    coef_ref[1:2, 1:_H] = g
    coef_ref[2:3, 0:1] = b0
    tip_full = tip_ref[...]
    for i in range(_N // _RBLK):
        blk = ts[:, i * _RBLK:(i + 1) * _RBLK]
        coef_ref[8:16, i:i + 1] = jnp.min(blk, axis=1, keepdims=True)
        coef_ref[8:16, 8 + i:9 + i] = jnp.max(blk, axis=1, keepdims=True)
    for t in range(_N // _CBLK):
        blk = tip_full[:, t * _CBLK:(t + 1) * _CBLK]
        coef_ref[8:16, 16 + t:17 + t] = jnp.min(blk, axis=1, keepdims=True)
        coef_ref[8:16, 24 + t:25 + t] = jnp.max(blk, axis=1, keepdims=True)


def _band_body(coef_ref, tsc_ref, tip_ref, out_ref):
    b = pl.program_id(0)
    i = pl.program_id(1)
    u_knots = [coef_ref[0, k] for k in range(_H + 1)]
    g_coefs = [coef_ref[1, k] for k in range(_H)]
    b0 = coef_ref[2, 0]
    u_last = u_knots[_H]
    rmin = coef_ref[8 + b, i]
    rmax = coef_ref[8 + b, 8 + i]
    tsb = jnp.broadcast_to(tsc_ref[0], (_RBLK, _CBLK))
    nt = _N // _CBLK

    def _interp(ti):
        atau = jnp.abs(tsb - ti)
        accs = [b0 + g_coefs[0] * atau,
                g_coefs[1] * jnp.maximum(atau - u_knots[1], 0.0),
                g_coefs[2] * jnp.maximum(atau - u_knots[2], 0.0)]
        for k in range(3, _H):
            accs[k % 3] = accs[k % 3] + g_coefs[k] * jnp.maximum(
                atau - u_knots[k], 0.0)
        acc = (accs[0] + accs[1]) + accs[2]
        return jnp.where(atau > u_last, 0.0, acc)

    out_ref[0] = jnp.zeros((_RBLK, _N - 1), jnp.float32)

    lives = []
    for t in range(nt):
        cmin = coef_ref[8 + b, 16 + t]
        cmax = coef_ref[8 + b, 24 + t]
        lives.append(jnp.logical_and(rmax - cmin >= -u_last,
                                     rmin - cmax <= u_last))
    t_lo = jnp.int32(nt)
    t_hi = jnp.int32(0)
    for t in range(nt):
        t_lo = jnp.minimum(t_lo, jnp.where(lives[t], t, nt))
        t_hi = jnp.maximum(t_hi, jnp.where(lives[t], t + 1, 0))

    def _tile(t, _):
        lo = t * _CBLK
        ti = tip_ref[0, :, pl.ds(lo, _CBLK)]
        out_ref[0, :, pl.ds(lo, _CBLK)] = _interp(ti)
        return _

    lax.fori_loop(t_lo, jnp.minimum(t_hi, nt - 1), _tile, None)

    @pl.when(lives[nt - 1])
    def _tail():
        lo = (nt - 1) * _CBLK
        val = _interp(tip_ref[0, :, lo:lo + _CBLK])
        out_ref[0, :, lo:_N - 1] = val[:, :_CBLK - 1]


def kernel(velocity_mps, depth_axis_m, wavelet_time_s, wavelet_amp):
    v = velocity_mps.astype(jnp.float32)
    d = depth_axis_m.astype(jnp.float32).reshape(1, _N)
    wt = wavelet_time_s.astype(jnp.float32).reshape(1, _K)
    wa = wavelet_amp.astype(jnp.float32).reshape(1, _K)

    ts, tip, coef = pl.pallas_call(
        _twt_body,
        out_shape=(
            jax.ShapeDtypeStruct((_B, _N), jnp.float32),
            jax.ShapeDtypeStruct((_B, _N), jnp.float32),
            jax.ShapeDtypeStruct((16, 128), jnp.float32),
        ),
    )(v, d, wt, wa)

    tsc = ts.reshape(_B, _N, 1)
    tip3 = tip.reshape(_B, 1, _N)
    nr = _N // _RBLK

    return pl.pallas_call(
        _band_body,
        grid=(_B, nr),
        in_specs=[
            pl.BlockSpec(memory_space=pltpu.SMEM),
            pl.BlockSpec((1, _RBLK, 1), lambda b, i: (b, i, 0)),
            pl.BlockSpec((1, 1, _N), lambda b, i: (b, 0, 0)),
        ],
        out_specs=pl.BlockSpec((1, _RBLK, _N - 1), lambda b, i: (b, i, 0)),
        out_shape=jax.ShapeDtypeStruct((_B, _N, _N - 1), jnp.float32),
        compiler_params=pltpu.CompilerParams(
            dimension_semantics=("arbitrary", "arbitrary")),
    )(coef, tsc, tip3)
